# Initial kernel scaffold; baseline (speedup 1.0000x reference)
import jax
import jax.numpy as jnp
from jax.experimental import pallas as pl


def kernel(x, edge_index, edge_attr, dW0, dWe0, db0, dW1, dWe1, db1, dW2, dWe2, db2, p0, p1, uW0, uWe0, ub0, uW1, uWe1, ub1):
    raise NotImplementedError("write your pallas kernel here")


# speedup vs baseline: 13.1124x; 13.1124x over previous
import functools

import jax
import jax.numpy as jnp
from jax import lax
from jax.experimental import pallas as pl
from jax.experimental.pallas import tpu as pltpu
from jax.experimental.pallas import tpu_sc as plsc

NN = 10000
DD = 128
EE = 320000
WT = 144
NC = 2
NS = 16
NW = NC * NS
RPT = 624
TAIL = NN - NS * RPT
ZR = 16
CH = 80
EPW = EE // NW
NIT = EPW // CH


def _sc_agg_body(tab, srci, dsti, out, idx_s, idx_d, rows, zbuf, acc, sem):
    cid = lax.axis_index("c")
    sid = lax.axis_index("s")
    wid = sid * NC + cid

    zvec = jnp.zeros((16,), jnp.float32)

    def zfill(i, c):
        for j in range(WT // 16):
            zbuf[i, pl.ds(16 * j, 16)] = zvec
        return c

    lax.fori_loop(0, ZR, zfill, 0)

    def zcopy(t, c):
        pltpu.sync_copy(zbuf, acc.at[pl.ds(sid * RPT + t * ZR, ZR)])
        return c

    lax.fori_loop(0, RPT // ZR, zcopy, 0)

    @pl.when(sid == 0)
    def _():
        pltpu.sync_copy(zbuf.at[pl.ds(0, TAIL)], acc.at[pl.ds(NS * RPT, TAIL)])

    plsc.subcore_barrier()

    def step(i, c):
        base = wid * EPW + i * CH
        pltpu.sync_copy(srci.at[pl.ds(base, CH)], idx_s)
        pltpu.sync_copy(dsti.at[pl.ds(base, CH)], idx_d)
        pltpu.async_copy(tab.at[idx_s], rows, sem).wait()
        pltpu.sync_copy(rows, acc.at[idx_d], add=True)
        return c

    lax.fori_loop(0, NIT, step, 0)
    plsc.subcore_barrier()
    pltpu.sync_copy(acc.at[pl.ds(sid * RPT, RPT)],
                    out.at[cid, pl.ds(sid * RPT, RPT)])

    @pl.when(sid == 0)
    def _():
        pltpu.sync_copy(acc.at[pl.ds(NS * RPT, TAIL)],
                        out.at[cid, pl.ds(NS * RPT, TAIL)])


_sc_agg = functools.partial(
    pl.kernel,
    out_type=jax.ShapeDtypeStruct((NC, NN, WT), jnp.float32),
    mesh=plsc.VectorSubcoreMesh(core_axis_name="c", subcore_axis_name="s",
                                num_cores=NC),
    compiler_params=pltpu.CompilerParams(use_tc_tiling_on_sc=False),
    scratch_types=[
        pltpu.VMEM((CH,), jnp.int32),
        pltpu.VMEM((CH,), jnp.int32),
        pltpu.VMEM((CH, WT), jnp.float32),
        pltpu.VMEM((ZR, WT), jnp.float32),
        pltpu.VMEM_SHARED((NN, WT), jnp.float32),
        pltpu.SemaphoreType.DMA,
    ],
)(_sc_agg_body)


def _mm_build_body(x_ref, w_ref, m_ref, t_ref):
    h = jnp.dot(x_ref[...], w_ref[...], preferred_element_type=jnp.float32)
    t_ref[:, :DD] = h
    col = lax.broadcasted_iota(jnp.int32, (NN, WT - DD), 1)
    t_ref[:, DD:] = jnp.where(col == 0, m_ref[...], 0.0)


def _mm_build2_body(x_ref, r_ref, w_ref, m_ref, t_ref):
    h = jnp.dot(x_ref[...] + r_ref[...], w_ref[...],
                preferred_element_type=jnp.float32)
    t_ref[:, :DD] = h
    col = lax.broadcasted_iota(jnp.int32, (NN, WT - DD), 1)
    t_ref[:, DD:] = jnp.where(col == 0, m_ref[...], 0.0)


_T_SHAPE = jax.ShapeDtypeStruct((NN, WT), jnp.float32)
_X_SHAPE = jax.ShapeDtypeStruct((NN, DD), jnp.float32)
_mm_build = pl.pallas_call(_mm_build_body, out_shape=_T_SHAPE)
_mm_build2 = pl.pallas_call(_mm_build2_body, out_shape=_T_SHAPE)


def _psum(p_ref):
    a = p_ref[0, :, :DD]
    deg = p_ref[0, :, DD:DD + 1]
    for c in range(1, NC):
        a = a + p_ref[c, :, :DD]
        deg = deg + p_ref[c, :, DD:DD + 1]
    return a, deg + 1.0


def _combine_relu_body(p_ref, t_ref, b_ref, o_ref):
    a, deg = _psum(p_ref)
    o_ref[...] = jnp.maximum((a + t_ref[:, :DD]) / deg + b_ref[...], 0.0)


def _combine_relu_mask_body(p_ref, t_ref, b_ref, m_ref, o_ref):
    a, deg = _psum(p_ref)
    y = jnp.maximum((a + t_ref[:, :DD]) / deg + b_ref[...], 0.0)
    o_ref[...] = y * m_ref[...]


def _combine_final_body(p_ref, t_ref, b_ref, x_ref, o_ref):
    a, deg = _psum(p_ref)
    o_ref[...] = x_ref[...] + (a + t_ref[:, :DD]) / deg + b_ref[...]


_combine_relu = pl.pallas_call(_combine_relu_body, out_shape=_X_SHAPE)
_combine_relu_mask = pl.pallas_call(_combine_relu_mask_body, out_shape=_X_SHAPE)
_combine_final = pl.pallas_call(_combine_final_body, out_shape=_X_SHAPE)


def _score_body(x_ref, p_ref, s_ref):
    p = p_ref[...]
    nrm = jnp.sqrt(jnp.sum(p * p)) + 1e-16
    s_ref[...] = jnp.sum(x_ref[...] * p, axis=1, keepdims=True) / nrm


_score = pl.pallas_call(
    _score_body, out_shape=jax.ShapeDtypeStruct((NN, 1), jnp.float32))


def _pool_body(x_ref, s_ref, m_ref, kth_ref, xb_ref, mn_ref):
    s = s_ref[...]
    sel = (s >= kth_ref[0, 0]) & (m_ref[...] > 0.0)
    mn = sel.astype(jnp.float32)
    mn_ref[...] = mn
    xb_ref[...] = x_ref[...] * (jnp.tanh(s) * mn)


_pool = pl.pallas_call(
    _pool_body,
    out_shape=[_X_SHAPE, jax.ShapeDtypeStruct((NN, 1), jnp.float32)])


def kernel(x, edge_index, edge_attr, dW0, dWe0, db0, dW1, dWe1, db1,
           dW2, dWe2, db2, p0, p1, uW0, uWe0, ub0, uW1, uWe1, ub1):
    src = edge_index[0]
    dst = edge_index[1]
    ones = jnp.ones((NN, 1), jnp.float32)

    def conv(T, b, mode, mcol=None, xres=None):
        P = _sc_agg(T, src, dst)
        b2 = b.reshape(1, DD)
        if mode == "relu":
            return _combine_relu(P, T, b2)
        if mode == "relu_mask":
            return _combine_relu_mask(P, T, b2, mcol)
        return _combine_final(P, T, b2, xres)

    def pool(xin, mprev, p, k):
        s = _score(xin, p.reshape(1, DD))
        masked = jnp.where(mprev[:, 0] > 0, s[:, 0], -jnp.inf)
        kth = lax.top_k(masked, k)[0][k - 1].reshape(1, 1)
        return _pool(xin, s, mprev, kth)

    xa = conv(_mm_build(x, dW0, ones), db0, "relu")
    xb, m1 = pool(xa, ones, p0, 1000)
    xc = conv(_mm_build(xb, dW1, m1), db1, "relu_mask", m1)
    xd, m2 = pool(xc, m1, p1, 100)
    xe = conv(_mm_build(xd, dW2, m2), db2, "relu_mask", m2)
    xf = conv(_mm_build2(xe, xc, uW0, m1), ub0, "relu_mask", m1)
    return conv(_mm_build2(xf, xa, uW1, ones), ub1, "final", xres=x)


# speedup vs baseline: 25.5421x; 1.9479x over previous
import functools

import jax
import jax.numpy as jnp
from jax import lax
from jax.experimental import pallas as pl
from jax.experimental.pallas import tpu as pltpu
from jax.experimental.pallas import tpu_sc as plsc

NN = 10000
DD = 128
EE = 320000
WT = 144
NC = 2
NS = 16
NW = NC * NS
RPT = 624
TAIL = NN - NS * RPT
ZR = 8
CH = 128
NCK = EE // CH
CPW = NCK // NW
XTRA = NCK - CPW * NW


def _sc_agg_body(tab, eidx, out, idx, rows, zbuf, acc, gs0, gs1, ss0, ss1):
    cid = lax.axis_index("c")
    sid = lax.axis_index("s")
    wid = sid * NC + cid
    c0 = wid * CPW

    zvec = jnp.zeros((16,), jnp.float32)

    def zfill(i, c):
        for j in range(WT // 16):
            zbuf[i, pl.ds(16 * j, 16)] = zvec
        return c

    lax.fori_loop(0, ZR, zfill, 0)

    def zcopy(t, c):
        pltpu.sync_copy(zbuf, acc.at[pl.ds(sid * RPT + t * ZR, ZR)])
        return c

    lax.fori_loop(0, RPT // ZR, zcopy, 0)

    @pl.when(sid == 0)
    def _():
        pltpu.sync_copy(zbuf, acc.at[pl.ds(NS * RPT, ZR)])
        pltpu.sync_copy(zbuf, acc.at[pl.ds(NS * RPT + ZR, ZR)])

    plsc.subcore_barrier()

    gsem = (gs0, gs1)
    ssem = (ss0, ss1)

    def idx_load(i, b):
        pltpu.sync_copy(eidx.at[c0 + i], idx.at[b])

    def gstart(b):
        pltpu.async_copy(tab.at[idx.at[b, 0]], rows.at[b], gsem[b])

    def gwait(b):
        pltpu.make_async_copy(tab.at[idx.at[b, 0]], rows.at[b], gsem[b]).wait()

    def sstart(b):
        pltpu.async_copy(rows.at[b], acc.at[idx.at[b, 1]], ssem[b], add=True)

    def swait(b):
        pltpu.make_async_copy(rows.at[b], acc.at[idx.at[b, 1]],
                              ssem[b]).wait()

    idx_load(0, 0)
    gstart(0)
    idx_load(1, 1)
    gstart(1)
    gwait(0)
    sstart(0)

    def body(j, c):
        i0 = 2 * j
        swait(0)
        idx_load(i0, 0)
        gstart(0)
        gwait(1)
        sstart(1)
        swait(1)
        idx_load(i0 + 1, 1)
        gstart(1)
        gwait(0)
        sstart(0)
        return c

    lax.fori_loop(1, CPW // 2, body, 0)
    swait(0)
    gwait(1)
    sstart(1)
    swait(1)

    @pl.when(wid < XTRA)
    def _():
        pltpu.sync_copy(eidx.at[NW * CPW + wid], idx.at[0])
        pltpu.async_copy(tab.at[idx.at[0, 0]], rows.at[0], gs0).wait()
        pltpu.sync_copy(rows.at[0], acc.at[idx.at[0, 1]], add=True)

    plsc.subcore_barrier()
    pltpu.sync_copy(acc.at[pl.ds(sid * RPT, RPT)],
                    out.at[cid, pl.ds(sid * RPT, RPT)])

    @pl.when(sid == 0)
    def _():
        pltpu.sync_copy(acc.at[pl.ds(NS * RPT, TAIL)],
                        out.at[cid, pl.ds(NS * RPT, TAIL)])


_sc_agg = functools.partial(
    pl.kernel,
    out_type=jax.ShapeDtypeStruct((NC, NN, WT), jnp.float32),
    mesh=plsc.VectorSubcoreMesh(core_axis_name="c", subcore_axis_name="s",
                                num_cores=NC),
    compiler_params=pltpu.CompilerParams(use_tc_tiling_on_sc=False),
    scratch_types=[
        pltpu.VMEM((2, 2, CH), jnp.int32),
        pltpu.VMEM((2, CH, WT), jnp.float32),
        pltpu.VMEM((ZR, WT), jnp.float32),
        pltpu.VMEM_SHARED((NN, WT), jnp.float32),
        pltpu.SemaphoreType.DMA,
        pltpu.SemaphoreType.DMA,
        pltpu.SemaphoreType.DMA,
        pltpu.SemaphoreType.DMA,
    ],
)(_sc_agg_body)


def _mm_build_body(x_ref, w_ref, m_ref, t_ref):
    h = jnp.dot(x_ref[...], w_ref[...], preferred_element_type=jnp.float32)
    t_ref[:, :DD] = h
    col = lax.broadcasted_iota(jnp.int32, (NN, WT - DD), 1)
    t_ref[:, DD:] = jnp.where(col == 0, m_ref[...], 0.0)


def _mm_build2_body(x_ref, r_ref, w_ref, m_ref, t_ref):
    h = jnp.dot(x_ref[...] + r_ref[...], w_ref[...],
                preferred_element_type=jnp.float32)
    t_ref[:, :DD] = h
    col = lax.broadcasted_iota(jnp.int32, (NN, WT - DD), 1)
    t_ref[:, DD:] = jnp.where(col == 0, m_ref[...], 0.0)


_T_SHAPE = jax.ShapeDtypeStruct((NN, WT), jnp.float32)
_X_SHAPE = jax.ShapeDtypeStruct((NN, DD), jnp.float32)
_mm_build = pl.pallas_call(_mm_build_body, out_shape=_T_SHAPE)
_mm_build2 = pl.pallas_call(_mm_build2_body, out_shape=_T_SHAPE)


def _psum(p_ref):
    a = p_ref[0, :, :DD]
    deg = p_ref[0, :, DD:DD + 1]
    for c in range(1, NC):
        a = a + p_ref[c, :, :DD]
        deg = deg + p_ref[c, :, DD:DD + 1]
    return a, deg + 1.0


def _combine_relu_body(p_ref, t_ref, b_ref, o_ref):
    a, deg = _psum(p_ref)
    o_ref[...] = jnp.maximum((a + t_ref[:, :DD]) / deg + b_ref[...], 0.0)


def _combine_relu_mask_body(p_ref, t_ref, b_ref, m_ref, o_ref):
    a, deg = _psum(p_ref)
    y = jnp.maximum((a + t_ref[:, :DD]) / deg + b_ref[...], 0.0)
    o_ref[...] = y * m_ref[...]


def _combine_final_body(p_ref, t_ref, b_ref, x_ref, o_ref):
    a, deg = _psum(p_ref)
    o_ref[...] = x_ref[...] + (a + t_ref[:, :DD]) / deg + b_ref[...]


_combine_relu = pl.pallas_call(_combine_relu_body, out_shape=_X_SHAPE)
_combine_relu_mask = pl.pallas_call(_combine_relu_mask_body, out_shape=_X_SHAPE)
_combine_final = pl.pallas_call(_combine_final_body, out_shape=_X_SHAPE)


def _score_body(x_ref, p_ref, s_ref):
    p = p_ref[...]
    nrm = jnp.sqrt(jnp.sum(p * p)) + 1e-16
    s_ref[...] = jnp.sum(x_ref[...] * p, axis=1, keepdims=True) / nrm


_score = pl.pallas_call(
    _score_body, out_shape=jax.ShapeDtypeStruct((NN, 1), jnp.float32))


def _pool_body(x_ref, s_ref, m_ref, kth_ref, xb_ref, mn_ref):
    s = s_ref[...]
    sel = (s >= kth_ref[0, 0]) & (m_ref[...] > 0.0)
    mn = sel.astype(jnp.float32)
    mn_ref[...] = mn
    xb_ref[...] = x_ref[...] * (jnp.tanh(s) * mn)


_pool = pl.pallas_call(
    _pool_body,
    out_shape=[_X_SHAPE, jax.ShapeDtypeStruct((NN, 1), jnp.float32)])


def kernel(x, edge_index, edge_attr, dW0, dWe0, db0, dW1, dWe1, db1,
           dW2, dWe2, db2, p0, p1, uW0, uWe0, ub0, uW1, uWe1, ub1):
    eidx = jnp.transpose(edge_index.reshape(2, NCK, CH), (1, 0, 2))
    ones = jnp.ones((NN, 1), jnp.float32)

    def conv(T, b, mode, mcol=None, xres=None):
        P = _sc_agg(T, eidx)
        b2 = b.reshape(1, DD)
        if mode == "relu":
            return _combine_relu(P, T, b2)
        if mode == "relu_mask":
            return _combine_relu_mask(P, T, b2, mcol)
        return _combine_final(P, T, b2, xres)

    def pool(xin, mprev, p, k):
        s = _score(xin, p.reshape(1, DD))
        masked = jnp.where(mprev[:, 0] > 0, s[:, 0], -jnp.inf)
        kth = lax.top_k(masked, k)[0][k - 1].reshape(1, 1)
        return _pool(xin, s, mprev, kth)

    xa = conv(_mm_build(x, dW0, ones), db0, "relu")
    xb, m1 = pool(xa, ones, p0, 1000)
    xc = conv(_mm_build(xb, dW1, m1), db1, "relu_mask", m1)
    xd, m2 = pool(xc, m1, p1, 100)
    xe = conv(_mm_build(xd, dW2, m2), db2, "relu_mask", m2)
    xf = conv(_mm_build2(xe, xc, uW0, m1), ub0, "relu_mask", m1)
    return conv(_mm_build2(xf, xa, uW1, ones), ub1, "final", xres=x)


# speedup vs baseline: 26.5478x; 1.0394x over previous
import functools

import jax
import jax.numpy as jnp
from jax import lax
from jax.experimental import pallas as pl
from jax.experimental.pallas import tpu as pltpu
from jax.experimental.pallas import tpu_sc as plsc

NN = 10000
DD = 128
EE = 320000
WT = 144
NC = 2
NS = 16
NW = NC * NS
RPT = 624
TAIL = NN - NS * RPT
ZR = 8
CH = 128
NCK = EE // CH
CPW = NCK // NW
XTRA = NCK - CPW * NW


def _sc_agg_body(tab, eidx, out, idx, rows, zbuf, acc, gs0, gs1, ss0, ss1):
    cid = lax.axis_index("c")
    sid = lax.axis_index("s")
    wid = sid * NC + cid
    c0 = wid * CPW

    zvec = jnp.zeros((16,), jnp.float32)

    def zfill(i, c):
        for j in range(WT // 16):
            zbuf[i, pl.ds(16 * j, 16)] = zvec
        return c

    lax.fori_loop(0, ZR, zfill, 0)

    def zcopy(t, c):
        pltpu.sync_copy(zbuf, acc.at[pl.ds(sid * RPT + t * ZR, ZR)])
        return c

    lax.fori_loop(0, RPT // ZR, zcopy, 0)

    @pl.when(sid == 0)
    def _():
        pltpu.sync_copy(zbuf, acc.at[pl.ds(NS * RPT, ZR)])
        pltpu.sync_copy(zbuf, acc.at[pl.ds(NS * RPT + ZR, ZR)])

    plsc.subcore_barrier()

    gsem = (gs0, gs1)
    ssem = (ss0, ss1)

    def idx_load(i, b):
        pltpu.sync_copy(eidx.at[c0 + i], idx.at[b])

    def gstart(b):
        pltpu.async_copy(tab.at[idx.at[b, 0]], rows.at[b], gsem[b])

    def gwait(b):
        pltpu.make_async_copy(tab.at[idx.at[b, 0]], rows.at[b], gsem[b]).wait()

    def sstart(b):
        pltpu.async_copy(rows.at[b], acc.at[idx.at[b, 1]], ssem[b], add=True)

    def swait(b):
        pltpu.make_async_copy(rows.at[b], acc.at[idx.at[b, 1]],
                              ssem[b]).wait()

    idx_load(0, 0)
    gstart(0)
    idx_load(1, 1)
    gstart(1)
    gwait(0)
    sstart(0)

    def body(j, c):
        i0 = 2 * j
        swait(0)
        idx_load(i0, 0)
        gstart(0)
        gwait(1)
        sstart(1)
        swait(1)
        idx_load(i0 + 1, 1)
        gstart(1)
        gwait(0)
        sstart(0)
        return c

    lax.fori_loop(1, CPW // 2, body, 0)
    swait(0)
    gwait(1)
    sstart(1)
    swait(1)

    @pl.when(wid < XTRA)
    def _():
        pltpu.sync_copy(eidx.at[NW * CPW + wid], idx.at[0])
        pltpu.async_copy(tab.at[idx.at[0, 0]], rows.at[0], gs0).wait()
        pltpu.sync_copy(rows.at[0], acc.at[idx.at[0, 1]], add=True)

    plsc.subcore_barrier()
    pltpu.sync_copy(acc.at[pl.ds(sid * RPT, RPT)],
                    out.at[cid, pl.ds(sid * RPT, RPT)])

    @pl.when(sid == 0)
    def _():
        pltpu.sync_copy(acc.at[pl.ds(NS * RPT, TAIL)],
                        out.at[cid, pl.ds(NS * RPT, TAIL)])


_sc_agg = functools.partial(
    pl.kernel,
    out_type=jax.ShapeDtypeStruct((NC, NN, WT), jnp.float32),
    mesh=plsc.VectorSubcoreMesh(core_axis_name="c", subcore_axis_name="s",
                                num_cores=NC),
    compiler_params=pltpu.CompilerParams(use_tc_tiling_on_sc=False),
    scratch_types=[
        pltpu.VMEM((2, 2, CH), jnp.int32),
        pltpu.VMEM((2, CH, WT), jnp.float32),
        pltpu.VMEM((ZR, WT), jnp.float32),
        pltpu.VMEM_SHARED((NN, WT), jnp.float32),
        pltpu.SemaphoreType.DMA,
        pltpu.SemaphoreType.DMA,
        pltpu.SemaphoreType.DMA,
        pltpu.SemaphoreType.DMA,
    ],
)(_sc_agg_body)


_T_SHAPE = jax.ShapeDtypeStruct((NN, WT), jnp.float32)
_X_SHAPE = jax.ShapeDtypeStruct((NN, DD), jnp.float32)
_S_SHAPE = jax.ShapeDtypeStruct((NN, 1), jnp.float32)
_NEG_BIG = -3.0e38


def _mask_cols(m):
    col = lax.broadcasted_iota(jnp.int32, (NN, WT - DD), 1)
    return jnp.where(col == 0, m, 0.0)


def _mm_build_body(x_ref, w_ref, t_ref):
    h = jnp.dot(x_ref[...], w_ref[...], preferred_element_type=jnp.float32)
    t_ref[:, :DD] = h
    t_ref[:, DD:] = _mask_cols(jnp.ones((NN, 1), jnp.float32))


_mm_build = pl.pallas_call(_mm_build_body, out_shape=_T_SHAPE)


def _psum(p_ref):
    a = p_ref[0, :, :DD]
    deg = p_ref[0, :, DD:DD + 1]
    for c in range(1, NC):
        a = a + p_ref[c, :, :DD]
        deg = deg + p_ref[c, :, DD:DD + 1]
    return a, deg + 1.0


def _comb_score_body(p_ref, t_ref, b_ref, pv_ref, x_ref, s_ref, *, masked):
    a, deg = _psum(p_ref)
    y = jnp.maximum((a + t_ref[:, :DD]) / deg + b_ref[...], 0.0)
    if masked:
        m = t_ref[:, DD:DD + 1]
        y = y * m
    x_ref[...] = y
    pv = pv_ref[...]
    nrm = jnp.sqrt(jnp.sum(pv * pv)) + 1e-16
    s = jnp.sum(y * pv, axis=1, keepdims=True) / nrm
    if masked:
        s = jnp.where(m > 0, s, _NEG_BIG)
    s_ref[...] = s


_comb_score = pl.pallas_call(
    functools.partial(_comb_score_body, masked=False),
    out_shape=[_X_SHAPE, _S_SHAPE])
_comb_score_mask = pl.pallas_call(
    functools.partial(_comb_score_body, masked=True),
    out_shape=[_X_SHAPE, _S_SHAPE])


def _pool_mm_body(x_ref, s_ref, m_ref, kth_ref, w_ref, t_ref):
    s = s_ref[...]
    sel = (s >= kth_ref[0, 0]) & (m_ref[...] > 0.0)
    mn = sel.astype(jnp.float32)
    xb = x_ref[...] * (jnp.tanh(s) * mn)
    t_ref[:, :DD] = jnp.dot(xb, w_ref[...], preferred_element_type=jnp.float32)
    t_ref[:, DD:] = _mask_cols(mn)


_pool_mm = pl.pallas_call(_pool_mm_body, out_shape=_T_SHAPE)


def _comb_mm_res_body(p_ref, t_ref, b_ref, r_ref, w_ref, mn_ref, t2_ref):
    a, deg = _psum(p_ref)
    m = t_ref[:, DD:DD + 1]
    y = jnp.maximum((a + t_ref[:, :DD]) / deg + b_ref[...], 0.0) * m
    y = y + r_ref[...]
    t2_ref[:, :DD] = jnp.dot(y, w_ref[...], preferred_element_type=jnp.float32)
    t2_ref[:, DD:] = _mask_cols(mn_ref[...])


_comb_mm_res = pl.pallas_call(_comb_mm_res_body, out_shape=_T_SHAPE)


def _combine_final_body(p_ref, t_ref, b_ref, x_ref, o_ref):
    a, deg = _psum(p_ref)
    o_ref[...] = x_ref[...] + (a + t_ref[:, :DD]) / deg + b_ref[...]


_combine_final = pl.pallas_call(_combine_final_body, out_shape=_X_SHAPE)


def kernel(x, edge_index, edge_attr, dW0, dWe0, db0, dW1, dWe1, db1,
           dW2, dWe2, db2, p0, p1, uW0, uWe0, ub0, uW1, uWe1, ub1):
    eidx = jnp.transpose(edge_index.reshape(2, NCK, CH), (1, 0, 2))
    ones = jnp.ones((NN, 1), jnp.float32)

    def kth_of(s, k):
        return lax.top_k(s[:, 0], k)[0][k - 1].reshape(1, 1)

    T0 = _mm_build(x, dW0)
    P0 = _sc_agg(T0, eidx)
    xa, s0 = _comb_score(P0, T0, db0.reshape(1, DD), p0.reshape(1, DD))
    T1 = _pool_mm(xa, s0, ones, kth_of(s0, 1000), dW1)
    P1 = _sc_agg(T1, eidx)
    xc, s1 = _comb_score_mask(P1, T1, db1.reshape(1, DD), p1.reshape(1, DD))
    m1 = T1[:, DD:DD + 1]
    T2 = _pool_mm(xc, s1, m1, kth_of(s1, 100), dW2)
    P2 = _sc_agg(T2, eidx)
    T3 = _comb_mm_res(P2, T2, db2.reshape(1, DD), xc, uW0, m1)
    P3 = _sc_agg(T3, eidx)
    T4 = _comb_mm_res(P3, T3, ub0.reshape(1, DD), xa, uW1, ones)
    P4 = _sc_agg(T4, eidx)
    return _combine_final(P4, T4, ub1.reshape(1, DD), x)


# speedup vs baseline: 26.9556x; 1.0154x over previous
import functools

import jax
import jax.numpy as jnp
from jax import lax
from jax.experimental import pallas as pl
from jax.experimental.pallas import tpu as pltpu
from jax.experimental.pallas import tpu_sc as plsc

NN = 10000
DD = 128
EE = 320000
WT = 144
NC = 2
NS = 16
NW = NC * NS
RPT = 624
TAIL = NN - NS * RPT
ZR = 8
CH = 128
NCK = EE // CH
CPW = NCK // NW
XTRA = NCK - CPW * NW
CAP = (CPW + 1) * CH
DUMP = NN
ACR = NN + 8


def _zero_acc(sid, zbuf, acc):
    zvec = jnp.zeros((16,), jnp.float32)

    def zfill(i, c):
        for j in range(WT // 16):
            zbuf[i, pl.ds(16 * j, 16)] = zvec
        return c

    lax.fori_loop(0, ZR, zfill, 0)

    def zcopy(t, c):
        pltpu.sync_copy(zbuf, acc.at[pl.ds(sid * RPT + t * ZR, ZR)])
        return c

    lax.fori_loop(0, RPT // ZR, zcopy, 0)

    @pl.when(sid == 0)
    def _():
        pltpu.sync_copy(zbuf, acc.at[pl.ds(NS * RPT, ZR)])
        pltpu.sync_copy(zbuf, acc.at[pl.ds(NS * RPT + ZR, ZR)])


def _copy_out(cid, sid, acc, out):
    pltpu.sync_copy(acc.at[pl.ds(sid * RPT, RPT)],
                    out.at[cid, pl.ds(sid * RPT, RPT)])

    @pl.when(sid == 0)
    def _():
        pltpu.sync_copy(acc.at[pl.ds(NS * RPT, TAIL)],
                        out.at[cid, pl.ds(NS * RPT, TAIL)])


def _sc_agg_body(tab, eidx, out, idx, rows, zbuf, acc, gs0, gs1, ss0, ss1):
    cid = lax.axis_index("c")
    sid = lax.axis_index("s")
    wid = sid * NC + cid
    c0 = wid * CPW

    _zero_acc(sid, zbuf, acc)
    plsc.subcore_barrier()

    gsem = (gs0, gs1)
    ssem = (ss0, ss1)

    def idx_load(i, b):
        pltpu.sync_copy(eidx.at[c0 + i], idx.at[b])

    def gstart(b):
        pltpu.async_copy(tab.at[idx.at[b, 0]], rows.at[b], gsem[b])

    def gwait(b):
        pltpu.make_async_copy(tab.at[idx.at[b, 0]], rows.at[b], gsem[b]).wait()

    def sstart(b):
        pltpu.async_copy(rows.at[b], acc.at[idx.at[b, 1]], ssem[b], add=True)

    def swait(b):
        pltpu.make_async_copy(rows.at[b], acc.at[idx.at[b, 1]],
                              ssem[b]).wait()

    idx_load(0, 0)
    gstart(0)
    idx_load(1, 1)
    gstart(1)
    gwait(0)
    sstart(0)

    def body(j, c):
        i0 = 2 * j
        swait(0)
        idx_load(i0, 0)
        gstart(0)
        gwait(1)
        sstart(1)
        swait(1)
        idx_load(i0 + 1, 1)
        gstart(1)
        gwait(0)
        sstart(0)
        return c

    lax.fori_loop(1, CPW // 2, body, 0)
    swait(0)
    gwait(1)
    sstart(1)
    swait(1)

    @pl.when(wid < XTRA)
    def _():
        pltpu.sync_copy(eidx.at[NW * CPW + wid], idx.at[0])
        pltpu.async_copy(tab.at[idx.at[0, 0]], rows.at[0], gs0).wait()
        pltpu.sync_copy(rows.at[0], acc.at[idx.at[0, 1]], add=True)

    plsc.subcore_barrier()
    _copy_out(cid, sid, acc, out)


_sc_agg = functools.partial(
    pl.kernel,
    out_type=jax.ShapeDtypeStruct((NC, NN, WT), jnp.float32),
    mesh=plsc.VectorSubcoreMesh(core_axis_name="c", subcore_axis_name="s",
                                num_cores=NC),
    compiler_params=pltpu.CompilerParams(use_tc_tiling_on_sc=False, needs_layout_passes=False),
    scratch_types=[
        pltpu.VMEM((2, 2, CH), jnp.int32),
        pltpu.VMEM((2, CH, WT), jnp.float32),
        pltpu.VMEM((ZR, WT), jnp.float32),
        pltpu.VMEM_SHARED((NN, WT), jnp.float32),
        pltpu.SemaphoreType.DMA,
        pltpu.SemaphoreType.DMA,
        pltpu.SemaphoreType.DMA,
        pltpu.SemaphoreType.DMA,
    ],
)(_sc_agg_body)


def _dyn_agg_loop(tab, comp, wid, n, idxb, rows, acc, sem):
    def astep(i, c):
        pltpu.sync_copy(comp.at[wid, 0, pl.ds(i * CH, CH)], idxb.at[0])
        pltpu.sync_copy(comp.at[wid, 1, pl.ds(i * CH, CH)], idxb.at[1])
        pltpu.async_copy(tab.at[idxb.at[0]], rows, sem).wait()
        pltpu.sync_copy(rows, acc.at[idxb.at[1]], add=True)
        return c

    lax.fori_loop(0, n, astep, 0)


def _sc_cagg_body(tab, eidx, mvec, out, comp, cnt,
                  mv, stg, idxb, rows, zbuf, cntv, acc, sem):
    cid = lax.axis_index("c")
    sid = lax.axis_index("s")
    wid = sid * NC + cid
    c0 = wid * CPW

    _zero_acc(sid, zbuf, acc)
    pltpu.sync_copy(mvec, mv)
    iot = lax.iota(jnp.int32, 16)
    z16 = jnp.zeros((16,), jnp.int32)

    def group(s16, d16, c, lastck):
        ms = plsc.load_gather(mv, [s16])
        md = plsc.load_gather(mv, [d16])
        keep = (ms > 0.0) & (md > 0.0)
        ki = keep.astype(jnp.int32)
        incl = plsc.cumsum(ki)
        pos = c + incl - ki
        ring = (pos >> 7) & 1
        col = pos & 127
        plsc.store_scatter(stg, [ring, z16, col], s16, mask=keep)
        plsc.store_scatter(stg, [ring, z16 + 1, col], d16, mask=keep)
        c2 = c + jnp.max(incl)
        ck2 = c2 >> 7

        @pl.when(ck2 > lastck)
        def _():
            r = lastck & 1
            pltpu.sync_copy(stg.at[r, 0], comp.at[wid, 0, pl.ds(lastck * CH, CH)])
            pltpu.sync_copy(stg.at[r, 1], comp.at[wid, 1, pl.ds(lastck * CH, CH)])

        return c2, jnp.where(ck2 > lastck, lastck + 1, lastck)

    def do_chunk(c, lastck):
        for g in range(CH // 16):
            s16 = idxb[0, pl.ds(16 * g, 16)]
            d16 = idxb[1, pl.ds(16 * g, 16)]
            c, lastck = group(s16, d16, c, lastck)
        return c, lastck

    def chunk(i, carry):
        pltpu.sync_copy(eidx.at[c0 + i], idxb)
        return do_chunk(*carry)

    carry = lax.fori_loop(0, CPW, chunk, (jnp.int32(0), jnp.int32(0)))

    def chunk_x(i, carry):
        pltpu.sync_copy(eidx.at[NW * CPW + wid], idxb)
        return do_chunk(*carry)

    c, lastck = lax.fori_loop(0, jnp.where(wid < XTRA, 1, 0), chunk_x, carry)

    padn = (-c) & 127
    ck = c >> 7

    @pl.when(padn > 0)
    def _():
        for g in range(CH // 16):
            v = iot + 16 * g
            mk = v < padn
            pos = c + v
            ring = (pos >> 7) & 1
            col = pos & 127
            plsc.store_scatter(stg, [ring, z16, col], z16, mask=mk)
            plsc.store_scatter(stg, [ring, z16 + 1, col],
                               jnp.full((16,), DUMP, jnp.int32), mask=mk)
        r = ck & 1
        pltpu.sync_copy(stg.at[r, 0], comp.at[wid, 0, pl.ds(ck * CH, CH)])
        pltpu.sync_copy(stg.at[r, 1], comp.at[wid, 1, pl.ds(ck * CH, CH)])

    n = (c + 127) >> 7
    cntv[...] = jnp.full((16,), 0, jnp.int32) + n
    pltpu.sync_copy(cntv, cnt.at[wid])

    plsc.subcore_barrier()
    _dyn_agg_loop(tab, comp, wid, n, idxb, rows, acc, sem)
    plsc.subcore_barrier()
    _copy_out(cid, sid, acc, out)


_sc_cagg = functools.partial(
    pl.kernel,
    out_type=[
        jax.ShapeDtypeStruct((NC, NN, WT), jnp.float32),
        jax.ShapeDtypeStruct((NW, 2, CAP), jnp.int32),
        jax.ShapeDtypeStruct((NW, 16), jnp.int32),
    ],
    mesh=plsc.VectorSubcoreMesh(core_axis_name="c", subcore_axis_name="s",
                                num_cores=NC),
    compiler_params=pltpu.CompilerParams(use_tc_tiling_on_sc=False, needs_layout_passes=False),
    scratch_types=[
        pltpu.VMEM((NN,), jnp.float32),
        pltpu.VMEM((2, 2, CH), jnp.int32),
        pltpu.VMEM((2, CH), jnp.int32),
        pltpu.VMEM((CH, WT), jnp.float32),
        pltpu.VMEM((ZR, WT), jnp.float32),
        pltpu.VMEM((16,), jnp.int32),
        pltpu.VMEM_SHARED((ACR, WT), jnp.float32),
        pltpu.SemaphoreType.DMA,
    ],
)(_sc_cagg_body)


def _sc_dyn_body(tab, comp, cnt, out, idxb, rows, zbuf, cntv, acc, sem):
    cid = lax.axis_index("c")
    sid = lax.axis_index("s")
    wid = sid * NC + cid

    _zero_acc(sid, zbuf, acc)
    pltpu.sync_copy(cnt.at[wid], cntv)
    n = jnp.max(cntv[...])
    plsc.subcore_barrier()
    _dyn_agg_loop(tab, comp, wid, n, idxb, rows, acc, sem)
    plsc.subcore_barrier()
    _copy_out(cid, sid, acc, out)


_sc_dyn = functools.partial(
    pl.kernel,
    out_type=jax.ShapeDtypeStruct((NC, NN, WT), jnp.float32),
    mesh=plsc.VectorSubcoreMesh(core_axis_name="c", subcore_axis_name="s",
                                num_cores=NC),
    compiler_params=pltpu.CompilerParams(use_tc_tiling_on_sc=False, needs_layout_passes=False),
    scratch_types=[
        pltpu.VMEM((2, CH), jnp.int32),
        pltpu.VMEM((CH, WT), jnp.float32),
        pltpu.VMEM((ZR, WT), jnp.float32),
        pltpu.VMEM((16,), jnp.int32),
        pltpu.VMEM_SHARED((ACR, WT), jnp.float32),
        pltpu.SemaphoreType.DMA,
    ],
)(_sc_dyn_body)


_T_SHAPE = jax.ShapeDtypeStruct((NN, WT), jnp.float32)
_X_SHAPE = jax.ShapeDtypeStruct((NN, DD), jnp.float32)
_S_SHAPE = jax.ShapeDtypeStruct((NN, 1), jnp.float32)
_NEG_BIG = -3.0e38


def _mask_cols(m):
    col = lax.broadcasted_iota(jnp.int32, (NN, WT - DD), 1)
    return jnp.where(col == 0, m, 0.0)


def _mm_build_body(x_ref, w_ref, t_ref):
    h = jnp.dot(x_ref[...], w_ref[...], preferred_element_type=jnp.float32)
    t_ref[:, :DD] = h
    t_ref[:, DD:] = _mask_cols(jnp.ones((NN, 1), jnp.float32))


_mm_build = pl.pallas_call(_mm_build_body, out_shape=_T_SHAPE)


def _psum(p_ref):
    a = p_ref[0, :, :DD]
    deg = p_ref[0, :, DD:DD + 1]
    for c in range(1, NC):
        a = a + p_ref[c, :, :DD]
        deg = deg + p_ref[c, :, DD:DD + 1]
    return a, deg + 1.0


def _comb_score_body(p_ref, t_ref, b_ref, pv_ref, x_ref, s_ref, *, masked):
    a, deg = _psum(p_ref)
    y = jnp.maximum((a + t_ref[:, :DD]) / deg + b_ref[...], 0.0)
    if masked:
        m = t_ref[:, DD:DD + 1]
        y = y * m
    x_ref[...] = y
    pv = pv_ref[...]
    nrm = jnp.sqrt(jnp.sum(pv * pv)) + 1e-16
    s = jnp.sum(y * pv, axis=1, keepdims=True) / nrm
    if masked:
        s = jnp.where(m > 0, s, _NEG_BIG)
    s_ref[...] = s


_comb_score = pl.pallas_call(
    functools.partial(_comb_score_body, masked=False),
    out_shape=[_X_SHAPE, _S_SHAPE])
_comb_score_mask = pl.pallas_call(
    functools.partial(_comb_score_body, masked=True),
    out_shape=[_X_SHAPE, _S_SHAPE])


def _pool_mm_body(x_ref, s_ref, m_ref, kth_ref, w_ref, t_ref):
    s = s_ref[...]
    sel = (s >= kth_ref[0, 0]) & (m_ref[...] > 0.0)
    mn = sel.astype(jnp.float32)
    xb = x_ref[...] * (jnp.tanh(s) * mn)
    t_ref[:, :DD] = jnp.dot(xb, w_ref[...], preferred_element_type=jnp.float32)
    t_ref[:, DD:] = _mask_cols(mn)


_pool_mm = pl.pallas_call(_pool_mm_body, out_shape=_T_SHAPE)


def _comb_mm_res_body(p_ref, t_ref, b_ref, r_ref, w_ref, mn_ref, t2_ref):
    a, deg = _psum(p_ref)
    m = t_ref[:, DD:DD + 1]
    y = jnp.maximum((a + t_ref[:, :DD]) / deg + b_ref[...], 0.0) * m
    y = y + r_ref[...]
    t2_ref[:, :DD] = jnp.dot(y, w_ref[...], preferred_element_type=jnp.float32)
    t2_ref[:, DD:] = _mask_cols(mn_ref[...])


_comb_mm_res = pl.pallas_call(_comb_mm_res_body, out_shape=_T_SHAPE)


def _combine_final_body(p_ref, t_ref, b_ref, x_ref, o_ref):
    a, deg = _psum(p_ref)
    o_ref[...] = x_ref[...] + (a + t_ref[:, :DD]) / deg + b_ref[...]


_combine_final = pl.pallas_call(_combine_final_body, out_shape=_X_SHAPE)


def kernel(x, edge_index, edge_attr, dW0, dWe0, db0, dW1, dWe1, db1,
           dW2, dWe2, db2, p0, p1, uW0, uWe0, ub0, uW1, uWe1, ub1):
    eidx = jnp.transpose(edge_index.reshape(2, NCK, CH), (1, 0, 2))
    ones = jnp.ones((NN, 1), jnp.float32)

    def kth_of(s, k):
        return lax.top_k(s[:, 0], k)[0][k - 1].reshape(1, 1)

    T0 = _mm_build(x, dW0)
    P0 = _sc_agg(T0, eidx)
    xa, s0 = _comb_score(P0, T0, db0.reshape(1, DD), p0.reshape(1, DD))
    T1 = _pool_mm(xa, s0, ones, kth_of(s0, 1000), dW1)
    m1 = T1[:, DD:DD + 1]
    P1, comp1, cnt1 = _sc_cagg(T1, eidx, m1[:, 0])
    xc, s1 = _comb_score_mask(P1, T1, db1.reshape(1, DD), p1.reshape(1, DD))
    T2 = _pool_mm(xc, s1, m1, kth_of(s1, 100), dW2)
    P2, _, _ = _sc_cagg(T2, eidx, T2[:, DD])
    T3 = _comb_mm_res(P2, T2, db2.reshape(1, DD), xc, uW0, m1)
    P3 = _sc_dyn(T3, comp1, cnt1)
    T4 = _comb_mm_res(P3, T3, ub0.reshape(1, DD), xa, uW1, ones)
    P4 = _sc_agg(T4, eidx)
    return _combine_final(P4, T4, ub1.reshape(1, DD), x)


# speedup vs baseline: 28.3013x; 1.0499x over previous
import functools

import jax
import jax.numpy as jnp
from jax import lax
from jax.experimental import pallas as pl
from jax.experimental.pallas import tpu as pltpu
from jax.experimental.pallas import tpu_sc as plsc

NN = 10000
DD = 128
EE = 320000
WT = 144
NC = 2
NS = 16
NW = NC * NS
RPT = 624
TAIL = NN - NS * RPT
ZR = 8
CH = 128
NCK = EE // CH
CPW = NCK // NW
XTRA = NCK - CPW * NW
CAP = (CPW + 1) * CH
DUMP = NN
ACR = NN + 8


def _zero_acc(sid, zhbm, acc):
    pltpu.sync_copy(zhbm.at[pl.ds(sid * RPT, RPT)],
                    acc.at[pl.ds(sid * RPT, RPT)])

    @pl.when(sid == 0)
    def _():
        pltpu.sync_copy(zhbm.at[pl.ds(NS * RPT, TAIL)],
                        acc.at[pl.ds(NS * RPT, TAIL)])


def _copy_out(cid, sid, acc, out):
    pltpu.sync_copy(acc.at[pl.ds(sid * RPT, RPT)],
                    out.at[cid, pl.ds(sid * RPT, RPT)])

    @pl.when(sid == 0)
    def _():
        pltpu.sync_copy(acc.at[pl.ds(NS * RPT, TAIL)],
                        out.at[cid, pl.ds(NS * RPT, TAIL)])


def _sc_agg_body(tab, eidx, zhbm, out, idx, rows, acc, gs0, gs1, ss0, ss1):
    cid = lax.axis_index("c")
    sid = lax.axis_index("s")
    wid = sid * NC + cid
    c0 = wid * CPW

    _zero_acc(sid, zhbm, acc)
    plsc.subcore_barrier()

    gsem = (gs0, gs1)
    ssem = (ss0, ss1)

    def idx_load(i, b):
        pltpu.sync_copy(eidx.at[c0 + i], idx.at[b])

    def gstart(b):
        pltpu.async_copy(tab.at[idx.at[b, 0]], rows.at[b], gsem[b])

    def gwait(b):
        pltpu.make_async_copy(tab.at[idx.at[b, 0]], rows.at[b], gsem[b]).wait()

    def sstart(b):
        pltpu.async_copy(rows.at[b], acc.at[idx.at[b, 1]], ssem[b], add=True)

    def swait(b):
        pltpu.make_async_copy(rows.at[b], acc.at[idx.at[b, 1]],
                              ssem[b]).wait()

    idx_load(0, 0)
    gstart(0)
    idx_load(1, 1)
    gstart(1)
    gwait(0)
    sstart(0)

    def body(j, c):
        i0 = 2 * j
        swait(0)
        idx_load(i0, 0)
        gstart(0)
        gwait(1)
        sstart(1)
        swait(1)
        idx_load(i0 + 1, 1)
        gstart(1)
        gwait(0)
        sstart(0)
        return c

    lax.fori_loop(1, CPW // 2, body, 0)
    swait(0)
    gwait(1)
    sstart(1)
    swait(1)

    @pl.when(wid < XTRA)
    def _():
        pltpu.sync_copy(eidx.at[NW * CPW + wid], idx.at[0])
        pltpu.async_copy(tab.at[idx.at[0, 0]], rows.at[0], gs0).wait()
        pltpu.sync_copy(rows.at[0], acc.at[idx.at[0, 1]], add=True)

    plsc.subcore_barrier()
    _copy_out(cid, sid, acc, out)


_sc_agg = functools.partial(
    pl.kernel,
    out_type=jax.ShapeDtypeStruct((NC, NN, WT), jnp.float32),
    mesh=plsc.VectorSubcoreMesh(core_axis_name="c", subcore_axis_name="s",
                                num_cores=NC),
    compiler_params=pltpu.CompilerParams(use_tc_tiling_on_sc=False, needs_layout_passes=False),
    scratch_types=[
        pltpu.VMEM((2, 2, CH), jnp.int32),
        pltpu.VMEM((2, CH, WT), jnp.float32),
        pltpu.VMEM_SHARED((NN, WT), jnp.float32),
        pltpu.SemaphoreType.DMA,
        pltpu.SemaphoreType.DMA,
        pltpu.SemaphoreType.DMA,
        pltpu.SemaphoreType.DMA,
    ],
)(_sc_agg_body)


def _dyn_agg_loop(tab, comp, wid, n, idxb, rows, acc, sem):
    def astep(i, c):
        pltpu.sync_copy(comp.at[wid, 0, pl.ds(i * CH, CH)], idxb.at[0, 0])
        pltpu.sync_copy(comp.at[wid, 1, pl.ds(i * CH, CH)], idxb.at[0, 1])
        pltpu.async_copy(tab.at[idxb.at[0, 0]], rows, sem).wait()
        pltpu.sync_copy(rows, acc.at[idxb.at[0, 1]], add=True)
        return c

    lax.fori_loop(0, n, astep, 0)


def _sc_cagg_body(tab, eidx, mvec, zhbm, out, comp, cnt,
                  mv, stg, idxb, rows, cntv, acc, sem, is0, is1):
    cid = lax.axis_index("c")
    sid = lax.axis_index("s")
    wid = sid * NC + cid
    c0 = wid * CPW

    _zero_acc(sid, zhbm, acc)
    pltpu.sync_copy(mvec, mv)
    iot = lax.iota(jnp.int32, 16)
    z16 = jnp.zeros((16,), jnp.int32)
    isem = (is0, is1)

    def istart(i, b):
        pltpu.async_copy(eidx.at[c0 + i], idxb.at[b], isem[b])

    def iwait(i, b):
        pltpu.make_async_copy(eidx.at[c0 + i], idxb.at[b], isem[b]).wait()

    def group(s16, d16, c, lastck):
        ms = plsc.load_gather(mv, [s16])
        md = plsc.load_gather(mv, [d16])
        keep = (ms > 0.0) & (md > 0.0)
        ki = keep.astype(jnp.int32)
        incl = plsc.cumsum(ki)
        pos = c + incl - ki
        ring = (pos >> 7) & 1
        col = pos & 127
        plsc.store_scatter(stg, [ring, z16, col], s16, mask=keep)
        plsc.store_scatter(stg, [ring, z16 + 1, col], d16, mask=keep)
        c2 = c + jnp.max(incl)
        ck2 = c2 >> 7

        @pl.when(ck2 > lastck)
        def _():
            r = lastck & 1
            pltpu.sync_copy(stg.at[r, 0], comp.at[wid, 0, pl.ds(lastck * CH, CH)])
            pltpu.sync_copy(stg.at[r, 1], comp.at[wid, 1, pl.ds(lastck * CH, CH)])

        return c2, jnp.where(ck2 > lastck, lastck + 1, lastck)

    def do_chunk(b, c, lastck):
        for g in range(CH // 16):
            s16 = idxb[b, 0, pl.ds(16 * g, 16)]
            d16 = idxb[b, 1, pl.ds(16 * g, 16)]
            c, lastck = group(s16, d16, c, lastck)
        return c, lastck

    istart(0, 0)

    def pair(j, carry):
        for b in range(2):
            i = 2 * j + b
            iwait(i, b)

            @pl.when(i + 1 < CPW)
            def _():
                istart(i + 1, b ^ 1)

            carry = do_chunk(b, *carry)
        return carry

    carry = lax.fori_loop(0, CPW // 2, pair, (jnp.int32(0), jnp.int32(0)))

    def chunk_x(i, carry):
        pltpu.sync_copy(eidx.at[NW * CPW + wid], idxb.at[0])
        return do_chunk(0, *carry)

    c, lastck = lax.fori_loop(0, jnp.where(wid < XTRA, 1, 0), chunk_x, carry)

    padn = (-c) & 127
    ck = c >> 7

    @pl.when(padn > 0)
    def _():
        for g in range(CH // 16):
            v = iot + 16 * g
            mk = v < padn
            pos = c + v
            ring = (pos >> 7) & 1
            col = pos & 127
            plsc.store_scatter(stg, [ring, z16, col], z16, mask=mk)
            plsc.store_scatter(stg, [ring, z16 + 1, col],
                               jnp.full((16,), DUMP, jnp.int32), mask=mk)
        r = ck & 1
        pltpu.sync_copy(stg.at[r, 0], comp.at[wid, 0, pl.ds(ck * CH, CH)])
        pltpu.sync_copy(stg.at[r, 1], comp.at[wid, 1, pl.ds(ck * CH, CH)])

    n = (c + 127) >> 7
    cntv[...] = jnp.full((16,), 0, jnp.int32) + n
    pltpu.sync_copy(cntv, cnt.at[wid])

    plsc.subcore_barrier()
    _dyn_agg_loop(tab, comp, wid, n, idxb, rows, acc, sem)
    plsc.subcore_barrier()
    _copy_out(cid, sid, acc, out)


_sc_cagg = functools.partial(
    pl.kernel,
    out_type=[
        jax.ShapeDtypeStruct((NC, NN, WT), jnp.float32),
        jax.ShapeDtypeStruct((NW, 2, CAP), jnp.int32),
        jax.ShapeDtypeStruct((NW, 16), jnp.int32),
    ],
    mesh=plsc.VectorSubcoreMesh(core_axis_name="c", subcore_axis_name="s",
                                num_cores=NC),
    compiler_params=pltpu.CompilerParams(use_tc_tiling_on_sc=False, needs_layout_passes=False),
    scratch_types=[
        pltpu.VMEM((NN,), jnp.float32),
        pltpu.VMEM((2, 2, CH), jnp.int32),
        pltpu.VMEM((2, 2, CH), jnp.int32),
        pltpu.VMEM((CH, WT), jnp.float32),
        pltpu.VMEM((16,), jnp.int32),
        pltpu.VMEM_SHARED((ACR, WT), jnp.float32),
        pltpu.SemaphoreType.DMA,
        pltpu.SemaphoreType.DMA,
        pltpu.SemaphoreType.DMA,
    ],
)(_sc_cagg_body)


def _sc_dyn_body(tab, comp, cnt, zhbm, out, idxb, rows, cntv, acc, sem):
    cid = lax.axis_index("c")
    sid = lax.axis_index("s")
    wid = sid * NC + cid

    _zero_acc(sid, zhbm, acc)
    pltpu.sync_copy(cnt.at[wid], cntv)
    n = jnp.max(cntv[...])
    plsc.subcore_barrier()
    _dyn_agg_loop(tab, comp, wid, n, idxb, rows, acc, sem)
    plsc.subcore_barrier()
    _copy_out(cid, sid, acc, out)


_sc_dyn = functools.partial(
    pl.kernel,
    out_type=jax.ShapeDtypeStruct((NC, NN, WT), jnp.float32),
    mesh=plsc.VectorSubcoreMesh(core_axis_name="c", subcore_axis_name="s",
                                num_cores=NC),
    compiler_params=pltpu.CompilerParams(use_tc_tiling_on_sc=False, needs_layout_passes=False),
    scratch_types=[
        pltpu.VMEM((2, 2, CH), jnp.int32),
        pltpu.VMEM((CH, WT), jnp.float32),
        pltpu.VMEM((16,), jnp.int32),
        pltpu.VMEM_SHARED((ACR, WT), jnp.float32),
        pltpu.SemaphoreType.DMA,
    ],
)(_sc_dyn_body)


_T_SHAPE = jax.ShapeDtypeStruct((NN, WT), jnp.float32)
_X_SHAPE = jax.ShapeDtypeStruct((NN, DD), jnp.float32)
_S_SHAPE = jax.ShapeDtypeStruct((NN, 1), jnp.float32)
_NEG_BIG = -3.0e38


def _mask_cols(m):
    col = lax.broadcasted_iota(jnp.int32, (NN, WT - DD), 1)
    return jnp.where(col == 0, m, 0.0)


def _mm_build_body(x_ref, w_ref, t_ref):
    h = jnp.dot(x_ref[...], w_ref[...], preferred_element_type=jnp.float32)
    t_ref[:, :DD] = h
    t_ref[:, DD:] = _mask_cols(jnp.ones((NN, 1), jnp.float32))


_mm_build = pl.pallas_call(_mm_build_body, out_shape=_T_SHAPE)


def _psum(p_ref):
    a = p_ref[0, :, :DD]
    deg = p_ref[0, :, DD:DD + 1]
    for c in range(1, NC):
        a = a + p_ref[c, :, :DD]
        deg = deg + p_ref[c, :, DD:DD + 1]
    return a, deg + 1.0


def _comb_score_body(p_ref, t_ref, b_ref, pv_ref, x_ref, s_ref, *, masked):
    a, deg = _psum(p_ref)
    y = jnp.maximum((a + t_ref[:, :DD]) / deg + b_ref[...], 0.0)
    if masked:
        m = t_ref[:, DD:DD + 1]
        y = y * m
    x_ref[...] = y
    pv = pv_ref[...]
    nrm = jnp.sqrt(jnp.sum(pv * pv)) + 1e-16
    s = jnp.sum(y * pv, axis=1, keepdims=True) / nrm
    if masked:
        s = jnp.where(m > 0, s, _NEG_BIG)
    s_ref[...] = s


_comb_score = pl.pallas_call(
    functools.partial(_comb_score_body, masked=False),
    out_shape=[_X_SHAPE, _S_SHAPE])
_comb_score_mask = pl.pallas_call(
    functools.partial(_comb_score_body, masked=True),
    out_shape=[_X_SHAPE, _S_SHAPE])


def _pool_mm_body(x_ref, s_ref, m_ref, kth_ref, w_ref, t_ref):
    s = s_ref[...]
    sel = (s >= kth_ref[0, 0]) & (m_ref[...] > 0.0)
    mn = sel.astype(jnp.float32)
    xb = x_ref[...] * (jnp.tanh(s) * mn)
    t_ref[:, :DD] = jnp.dot(xb, w_ref[...], preferred_element_type=jnp.float32)
    t_ref[:, DD:] = _mask_cols(mn)


_pool_mm = pl.pallas_call(_pool_mm_body, out_shape=_T_SHAPE)


def _comb_mm_res_body(p_ref, t_ref, b_ref, r_ref, w_ref, mn_ref, t2_ref):
    a, deg = _psum(p_ref)
    m = t_ref[:, DD:DD + 1]
    y = jnp.maximum((a + t_ref[:, :DD]) / deg + b_ref[...], 0.0) * m
    y = y + r_ref[...]
    t2_ref[:, :DD] = jnp.dot(y, w_ref[...], preferred_element_type=jnp.float32)
    t2_ref[:, DD:] = _mask_cols(mn_ref[...])


_comb_mm_res = pl.pallas_call(_comb_mm_res_body, out_shape=_T_SHAPE)


def _combine_final_body(p_ref, t_ref, b_ref, x_ref, o_ref):
    a, deg = _psum(p_ref)
    o_ref[...] = x_ref[...] + (a + t_ref[:, :DD]) / deg + b_ref[...]


_combine_final = pl.pallas_call(_combine_final_body, out_shape=_X_SHAPE)


def kernel(x, edge_index, edge_attr, dW0, dWe0, db0, dW1, dWe1, db1,
           dW2, dWe2, db2, p0, p1, uW0, uWe0, ub0, uW1, uWe1, ub1):
    eidx = jnp.transpose(edge_index.reshape(2, NCK, CH), (1, 0, 2))
    ones = jnp.ones((NN, 1), jnp.float32)
    zeros = jnp.zeros((NN, WT), jnp.float32)

    def kth_of(s, k):
        return lax.top_k(s[:, 0], k)[0][k - 1].reshape(1, 1)

    T0 = _mm_build(x, dW0)
    P0 = _sc_agg(T0, eidx, zeros)
    xa, s0 = _comb_score(P0, T0, db0.reshape(1, DD), p0.reshape(1, DD))
    T1 = _pool_mm(xa, s0, ones, kth_of(s0, 1000), dW1)
    m1 = T1[:, DD:DD + 1]
    P1, comp1, cnt1 = _sc_cagg(T1, eidx, m1[:, 0], zeros)
    xc, s1 = _comb_score_mask(P1, T1, db1.reshape(1, DD), p1.reshape(1, DD))
    T2 = _pool_mm(xc, s1, m1, kth_of(s1, 100), dW2)
    P2, _, _ = _sc_cagg(T2, eidx, T2[:, DD], zeros)
    T3 = _comb_mm_res(P2, T2, db2.reshape(1, DD), xc, uW0, m1)
    P3 = _sc_dyn(T3, comp1, cnt1, zeros)
    T4 = _comb_mm_res(P3, T3, ub0.reshape(1, DD), xa, uW1, ones)
    P4 = _sc_agg(T4, eidx, zeros)
    return _combine_final(P4, T4, ub1.reshape(1, DD), x)


# speedup vs baseline: 29.5114x; 1.0428x over previous
import functools

import jax
import jax.numpy as jnp
from jax import lax
from jax.experimental import pallas as pl
from jax.experimental.pallas import tpu as pltpu
from jax.experimental.pallas import tpu_sc as plsc

NN = 10000
DD = 128
EE = 320000
WT = 144
NC = 2
NS = 16
NW = NC * NS
RPT = 624
TAIL = NN - NS * RPT
ZR = 8
CH = 128
NCK = EE // CH
CPW = NCK // NW
XTRA = NCK - CPW * NW
CAP = (CPW + 1) * CH
DUMP = NN
ACR = NN + 8


def _zero_acc(sid, zhbm, acc):
    pltpu.sync_copy(zhbm.at[pl.ds(sid * RPT, RPT)],
                    acc.at[pl.ds(sid * RPT, RPT)])

    @pl.when(sid == 0)
    def _():
        pltpu.sync_copy(zhbm.at[pl.ds(NS * RPT, TAIL)],
                        acc.at[pl.ds(NS * RPT, TAIL)])


def _copy_out(cid, sid, acc, out):
    pltpu.sync_copy(acc.at[pl.ds(sid * RPT, RPT)],
                    out.at[cid, pl.ds(sid * RPT, RPT)])

    @pl.when(sid == 0)
    def _():
        pltpu.sync_copy(acc.at[pl.ds(NS * RPT, TAIL)],
                        out.at[cid, pl.ds(NS * RPT, TAIL)])


def _sc_agg_body(tab, eidx, zhbm, out, idx, rows, acc, gs0, gs1, ss0, ss1):
    cid = lax.axis_index("c")
    sid = lax.axis_index("s")
    wid = sid * NC + cid
    c0 = wid * CPW

    _zero_acc(sid, zhbm, acc)
    plsc.subcore_barrier()

    gsem = (gs0, gs1)
    ssem = (ss0, ss1)

    def idx_load(i, b):
        pltpu.sync_copy(eidx.at[c0 + i], idx.at[b])

    def gstart(b):
        pltpu.async_copy(tab.at[idx.at[b, 0]], rows.at[b], gsem[b])

    def gwait(b):
        pltpu.make_async_copy(tab.at[idx.at[b, 0]], rows.at[b], gsem[b]).wait()

    def sstart(b):
        pltpu.async_copy(rows.at[b], acc.at[idx.at[b, 1]], ssem[b], add=True)

    def swait(b):
        pltpu.make_async_copy(rows.at[b], acc.at[idx.at[b, 1]],
                              ssem[b]).wait()

    idx_load(0, 0)
    gstart(0)
    idx_load(1, 1)
    gstart(1)
    gwait(0)
    sstart(0)

    def body(j, c):
        i0 = 2 * j
        swait(0)
        idx_load(i0, 0)
        gstart(0)
        gwait(1)
        sstart(1)
        swait(1)
        idx_load(i0 + 1, 1)
        gstart(1)
        gwait(0)
        sstart(0)
        return c

    lax.fori_loop(1, CPW // 2, body, 0)
    swait(0)
    gwait(1)
    sstart(1)
    swait(1)

    @pl.when(wid < XTRA)
    def _():
        pltpu.sync_copy(eidx.at[NW * CPW + wid], idx.at[0])
        pltpu.async_copy(tab.at[idx.at[0, 0]], rows.at[0], gs0).wait()
        pltpu.sync_copy(rows.at[0], acc.at[idx.at[0, 1]], add=True)

    plsc.subcore_barrier()
    _copy_out(cid, sid, acc, out)


_sc_agg = functools.partial(
    pl.kernel,
    out_type=jax.ShapeDtypeStruct((NC, NN, WT), jnp.float32),
    mesh=plsc.VectorSubcoreMesh(core_axis_name="c", subcore_axis_name="s",
                                num_cores=NC),
    compiler_params=pltpu.CompilerParams(use_tc_tiling_on_sc=False, needs_layout_passes=False),
    scratch_types=[
        pltpu.VMEM((2, 2, CH), jnp.int32),
        pltpu.VMEM((2, CH, WT), jnp.float32),
        pltpu.VMEM_SHARED((NN, WT), jnp.float32),
        pltpu.SemaphoreType.DMA,
        pltpu.SemaphoreType.DMA,
        pltpu.SemaphoreType.DMA,
        pltpu.SemaphoreType.DMA,
    ],
)(_sc_agg_body)


def _dyn_agg_loop(tab, comp, wid, n, idxb, rows, acc, sem):
    def astep(i, c):
        pltpu.sync_copy(comp.at[wid, 0, pl.ds(i * CH, CH)], idxb.at[0, 0])
        pltpu.sync_copy(comp.at[wid, 1, pl.ds(i * CH, CH)], idxb.at[0, 1])
        pltpu.async_copy(tab.at[idxb.at[0, 0]], rows, sem).wait()
        pltpu.sync_copy(rows, acc.at[idxb.at[0, 1]], add=True)
        return c

    lax.fori_loop(0, n, astep, 0)


def _sc_cagg_body(tab, eidx, mvec, zhbm, out, comp, cnt,
                  mv, stg, idxb, rows, cntv, acc, sem, is0, is1):
    cid = lax.axis_index("c")
    sid = lax.axis_index("s")
    wid = sid * NC + cid
    c0 = wid * CPW

    _zero_acc(sid, zhbm, acc)
    pltpu.sync_copy(mvec, mv)
    iot = lax.iota(jnp.int32, 16)
    z16 = jnp.zeros((16,), jnp.int32)
    isem = (is0, is1)

    def istart(i, b):
        pltpu.async_copy(eidx.at[c0 + i], idxb.at[b], isem[b])

    def iwait(i, b):
        pltpu.make_async_copy(eidx.at[c0 + i], idxb.at[b], isem[b]).wait()

    def group(s16, d16, c, lastck):
        ms = plsc.load_gather(mv, [s16])
        md = plsc.load_gather(mv, [d16])
        keep = (ms > 0.0) & (md > 0.0)
        ki = keep.astype(jnp.int32)
        incl = plsc.cumsum(ki)
        pos = c + incl - ki
        ring = (pos >> 7) & 1
        col = pos & 127
        plsc.store_scatter(stg, [ring, z16, col], s16, mask=keep)
        plsc.store_scatter(stg, [ring, z16 + 1, col], d16, mask=keep)
        c2 = c + jnp.max(incl)
        ck2 = c2 >> 7

        @pl.when(ck2 > lastck)
        def _():
            r = lastck & 1
            pltpu.sync_copy(stg.at[r, 0], comp.at[wid, 0, pl.ds(lastck * CH, CH)])
            pltpu.sync_copy(stg.at[r, 1], comp.at[wid, 1, pl.ds(lastck * CH, CH)])

        return c2, jnp.where(ck2 > lastck, lastck + 1, lastck)

    def do_chunk(b, c, lastck):
        for g in range(CH // 16):
            s16 = idxb[b, 0, pl.ds(16 * g, 16)]
            d16 = idxb[b, 1, pl.ds(16 * g, 16)]
            c, lastck = group(s16, d16, c, lastck)
        return c, lastck

    istart(0, 0)

    def pair(j, carry):
        for b in range(2):
            i = 2 * j + b
            iwait(i, b)

            @pl.when(i + 1 < CPW)
            def _():
                istart(i + 1, b ^ 1)

            carry = do_chunk(b, *carry)
        return carry

    carry = lax.fori_loop(0, CPW // 2, pair, (jnp.int32(0), jnp.int32(0)))

    def chunk_x(i, carry):
        pltpu.sync_copy(eidx.at[NW * CPW + wid], idxb.at[0])
        return do_chunk(0, *carry)

    c, lastck = lax.fori_loop(0, jnp.where(wid < XTRA, 1, 0), chunk_x, carry)

    padn = (-c) & 127
    ck = c >> 7

    @pl.when(padn > 0)
    def _():
        for g in range(CH // 16):
            v = iot + 16 * g
            mk = v < padn
            pos = c + v
            ring = (pos >> 7) & 1
            col = pos & 127
            plsc.store_scatter(stg, [ring, z16, col], z16, mask=mk)
            plsc.store_scatter(stg, [ring, z16 + 1, col],
                               jnp.full((16,), DUMP, jnp.int32), mask=mk)
        r = ck & 1
        pltpu.sync_copy(stg.at[r, 0], comp.at[wid, 0, pl.ds(ck * CH, CH)])
        pltpu.sync_copy(stg.at[r, 1], comp.at[wid, 1, pl.ds(ck * CH, CH)])

    n = (c + 127) >> 7
    cntv[...] = jnp.full((16,), 0, jnp.int32) + n
    pltpu.sync_copy(cntv, cnt.at[wid])

    plsc.subcore_barrier()
    _dyn_agg_loop(tab, comp, wid, n, idxb, rows, acc, sem)
    plsc.subcore_barrier()
    _copy_out(cid, sid, acc, out)


_sc_cagg = functools.partial(
    pl.kernel,
    out_type=[
        jax.ShapeDtypeStruct((NC, NN, WT), jnp.float32),
        jax.ShapeDtypeStruct((NW, 2, CAP), jnp.int32),
        jax.ShapeDtypeStruct((NW, 16), jnp.int32),
    ],
    mesh=plsc.VectorSubcoreMesh(core_axis_name="c", subcore_axis_name="s",
                                num_cores=NC),
    compiler_params=pltpu.CompilerParams(use_tc_tiling_on_sc=False, needs_layout_passes=False),
    scratch_types=[
        pltpu.VMEM((NN,), jnp.float32),
        pltpu.VMEM((2, 2, CH), jnp.int32),
        pltpu.VMEM((2, 2, CH), jnp.int32),
        pltpu.VMEM((CH, WT), jnp.float32),
        pltpu.VMEM((16,), jnp.int32),
        pltpu.VMEM_SHARED((ACR, WT), jnp.float32),
        pltpu.SemaphoreType.DMA,
        pltpu.SemaphoreType.DMA,
        pltpu.SemaphoreType.DMA,
    ],
)(_sc_cagg_body)


def _sc_cagg2_body(tab, comp1, cnt1, mvec, zhbm, out, comp, cnt,
                   mv, stg, idxb, rows, cntv, acc, sem):
    cid = lax.axis_index("c")
    sid = lax.axis_index("s")
    wid = sid * NC + cid

    _zero_acc(sid, zhbm, acc)
    pltpu.sync_copy(mvec, mv)
    pltpu.sync_copy(cnt1.at[wid], cntv)
    n1 = jnp.max(cntv[...])
    iot = lax.iota(jnp.int32, 16)
    z16 = jnp.zeros((16,), jnp.int32)

    def group(s16, d16, c, lastck):
        ms = plsc.load_gather(mv, [s16])
        md = plsc.load_gather(mv, [d16])
        keep = (ms > 0.0) & (md > 0.0)
        ki = keep.astype(jnp.int32)
        incl = plsc.cumsum(ki)
        pos = c + incl - ki
        ring = (pos >> 7) & 1
        col = pos & 127
        plsc.store_scatter(stg, [ring, z16, col], s16, mask=keep)
        plsc.store_scatter(stg, [ring, z16 + 1, col], d16, mask=keep)
        c2 = c + jnp.max(incl)
        ck2 = c2 >> 7

        @pl.when(ck2 > lastck)
        def _():
            r = lastck & 1
            pltpu.sync_copy(stg.at[r, 0], comp.at[wid, 0, pl.ds(lastck * CH, CH)])
            pltpu.sync_copy(stg.at[r, 1], comp.at[wid, 1, pl.ds(lastck * CH, CH)])

        return c2, jnp.where(ck2 > lastck, lastck + 1, lastck)

    def chunk(i, carry):
        pltpu.sync_copy(comp1.at[wid, 0, pl.ds(i * CH, CH)], idxb.at[0, 0])
        pltpu.sync_copy(comp1.at[wid, 1, pl.ds(i * CH, CH)], idxb.at[0, 1])
        c, lastck = carry
        for g in range(CH // 16):
            s16 = idxb[0, 0, pl.ds(16 * g, 16)]
            d16 = idxb[0, 1, pl.ds(16 * g, 16)]
            c, lastck = group(s16, d16, c, lastck)
        return c, lastck

    c, lastck = lax.fori_loop(0, n1, chunk, (jnp.int32(0), jnp.int32(0)))

    padn = (-c) & 127
    ck = c >> 7

    @pl.when(padn > 0)
    def _():
        for g in range(CH // 16):
            v = iot + 16 * g
            mk = v < padn
            pos = c + v
            ring = (pos >> 7) & 1
            col = pos & 127
            plsc.store_scatter(stg, [ring, z16, col], z16, mask=mk)
            plsc.store_scatter(stg, [ring, z16 + 1, col],
                               jnp.full((16,), DUMP, jnp.int32), mask=mk)
        r = ck & 1
        pltpu.sync_copy(stg.at[r, 0], comp.at[wid, 0, pl.ds(ck * CH, CH)])
        pltpu.sync_copy(stg.at[r, 1], comp.at[wid, 1, pl.ds(ck * CH, CH)])

    n = (c + 127) >> 7
    cntv[...] = jnp.full((16,), 0, jnp.int32) + n
    pltpu.sync_copy(cntv, cnt.at[wid])

    plsc.subcore_barrier()
    _dyn_agg_loop(tab, comp, wid, n, idxb, rows, acc, sem)
    plsc.subcore_barrier()
    _copy_out(cid, sid, acc, out)


_sc_cagg2 = functools.partial(
    pl.kernel,
    out_type=[
        jax.ShapeDtypeStruct((NC, NN, WT), jnp.float32),
        jax.ShapeDtypeStruct((NW, 2, CAP), jnp.int32),
        jax.ShapeDtypeStruct((NW, 16), jnp.int32),
    ],
    mesh=plsc.VectorSubcoreMesh(core_axis_name="c", subcore_axis_name="s",
                                num_cores=NC),
    compiler_params=pltpu.CompilerParams(use_tc_tiling_on_sc=False, needs_layout_passes=False),
    scratch_types=[
        pltpu.VMEM((NN,), jnp.float32),
        pltpu.VMEM((2, 2, CH), jnp.int32),
        pltpu.VMEM((2, 2, CH), jnp.int32),
        pltpu.VMEM((CH, WT), jnp.float32),
        pltpu.VMEM((16,), jnp.int32),
        pltpu.VMEM_SHARED((ACR, WT), jnp.float32),
        pltpu.SemaphoreType.DMA,
    ],
)(_sc_cagg2_body)


def _sc_dyn_body(tab, comp, cnt, zhbm, out, idxb, rows, cntv, acc, sem):
    cid = lax.axis_index("c")
    sid = lax.axis_index("s")
    wid = sid * NC + cid

    _zero_acc(sid, zhbm, acc)
    pltpu.sync_copy(cnt.at[wid], cntv)
    n = jnp.max(cntv[...])
    plsc.subcore_barrier()
    _dyn_agg_loop(tab, comp, wid, n, idxb, rows, acc, sem)
    plsc.subcore_barrier()
    _copy_out(cid, sid, acc, out)


_sc_dyn = functools.partial(
    pl.kernel,
    out_type=jax.ShapeDtypeStruct((NC, NN, WT), jnp.float32),
    mesh=plsc.VectorSubcoreMesh(core_axis_name="c", subcore_axis_name="s",
                                num_cores=NC),
    compiler_params=pltpu.CompilerParams(use_tc_tiling_on_sc=False, needs_layout_passes=False),
    scratch_types=[
        pltpu.VMEM((2, 2, CH), jnp.int32),
        pltpu.VMEM((CH, WT), jnp.float32),
        pltpu.VMEM((16,), jnp.int32),
        pltpu.VMEM_SHARED((ACR, WT), jnp.float32),
        pltpu.SemaphoreType.DMA,
    ],
)(_sc_dyn_body)


_T_SHAPE = jax.ShapeDtypeStruct((NN, WT), jnp.float32)
_X_SHAPE = jax.ShapeDtypeStruct((NN, DD), jnp.float32)
_S_SHAPE = jax.ShapeDtypeStruct((NN, 1), jnp.float32)
_NEG_BIG = -3.0e38


def _mask_cols(m):
    col = lax.broadcasted_iota(jnp.int32, (NN, WT - DD), 1)
    return jnp.where(col == 0, m, 0.0)


def _mm_build_body(x_ref, w_ref, t_ref):
    h = jnp.dot(x_ref[...], w_ref[...], preferred_element_type=jnp.float32)
    t_ref[:, :DD] = h
    t_ref[:, DD:] = _mask_cols(jnp.ones((NN, 1), jnp.float32))


_mm_build = pl.pallas_call(_mm_build_body, out_shape=_T_SHAPE)


def _psum(p_ref):
    a = p_ref[0, :, :DD]
    deg = p_ref[0, :, DD:DD + 1]
    for c in range(1, NC):
        a = a + p_ref[c, :, :DD]
        deg = deg + p_ref[c, :, DD:DD + 1]
    return a, deg + 1.0


def _comb_score_body(p_ref, t_ref, b_ref, pv_ref, x_ref, s_ref, *, masked):
    a, deg = _psum(p_ref)
    y = jnp.maximum((a + t_ref[:, :DD]) / deg + b_ref[...], 0.0)
    if masked:
        m = t_ref[:, DD:DD + 1]
        y = y * m
    x_ref[...] = y
    pv = pv_ref[...]
    nrm = jnp.sqrt(jnp.sum(pv * pv)) + 1e-16
    s = jnp.sum(y * pv, axis=1, keepdims=True) / nrm
    if masked:
        s = jnp.where(m > 0, s, _NEG_BIG)
    s_ref[...] = s


_comb_score = pl.pallas_call(
    functools.partial(_comb_score_body, masked=False),
    out_shape=[_X_SHAPE, _S_SHAPE])
_comb_score_mask = pl.pallas_call(
    functools.partial(_comb_score_body, masked=True),
    out_shape=[_X_SHAPE, _S_SHAPE])


def _pool_mm_body(x_ref, s_ref, m_ref, kth_ref, w_ref, t_ref):
    s = s_ref[...]
    sel = (s >= kth_ref[0, 0]) & (m_ref[...] > 0.0)
    mn = sel.astype(jnp.float32)
    xb = x_ref[...] * (jnp.tanh(s) * mn)
    t_ref[:, :DD] = jnp.dot(xb, w_ref[...], preferred_element_type=jnp.float32)
    t_ref[:, DD:] = _mask_cols(mn)


_pool_mm = pl.pallas_call(_pool_mm_body, out_shape=_T_SHAPE)


def _comb_mm_res_body(p_ref, t_ref, b_ref, r_ref, w_ref, mn_ref, t2_ref):
    a, deg = _psum(p_ref)
    m = t_ref[:, DD:DD + 1]
    y = jnp.maximum((a + t_ref[:, :DD]) / deg + b_ref[...], 0.0) * m
    y = y + r_ref[...]
    t2_ref[:, :DD] = jnp.dot(y, w_ref[...], preferred_element_type=jnp.float32)
    t2_ref[:, DD:] = _mask_cols(mn_ref[...])


_comb_mm_res = pl.pallas_call(_comb_mm_res_body, out_shape=_T_SHAPE)


def _combine_final_body(p_ref, t_ref, b_ref, x_ref, o_ref):
    a, deg = _psum(p_ref)
    o_ref[...] = x_ref[...] + (a + t_ref[:, :DD]) / deg + b_ref[...]


_combine_final = pl.pallas_call(_combine_final_body, out_shape=_X_SHAPE)


def kernel(x, edge_index, edge_attr, dW0, dWe0, db0, dW1, dWe1, db1,
           dW2, dWe2, db2, p0, p1, uW0, uWe0, ub0, uW1, uWe1, ub1):
    eidx = jnp.transpose(edge_index.reshape(2, NCK, CH), (1, 0, 2))
    ones = jnp.ones((NN, 1), jnp.float32)
    zeros = jnp.zeros((NN, WT), jnp.float32)

    def kth_of(s, k):
        return lax.top_k(s[:, 0], k)[0][k - 1].reshape(1, 1)

    T0 = _mm_build(x, dW0)
    P0 = _sc_agg(T0, eidx, zeros)
    xa, s0 = _comb_score(P0, T0, db0.reshape(1, DD), p0.reshape(1, DD))
    T1 = _pool_mm(xa, s0, ones, kth_of(s0, 1000), dW1)
    m1 = T1[:, DD:DD + 1]
    P1, comp1, cnt1 = _sc_cagg(T1, eidx, m1[:, 0], zeros)
    xc, s1 = _comb_score_mask(P1, T1, db1.reshape(1, DD), p1.reshape(1, DD))
    T2 = _pool_mm(xc, s1, m1, kth_of(s1, 100), dW2)
    P2, _, _ = _sc_cagg2(T2, comp1, cnt1, T2[:, DD], zeros)
    T3 = _comb_mm_res(P2, T2, db2.reshape(1, DD), xc, uW0, m1)
    P3 = _sc_dyn(T3, comp1, cnt1, zeros)
    T4 = _comb_mm_res(P3, T3, ub0.reshape(1, DD), xa, uW1, ones)
    P4 = _sc_agg(T4, eidx, zeros)
    return _combine_final(P4, T4, ub1.reshape(1, DD), x)
